# baseline (speedup 1.0000x reference)
import functools

import jax
import jax.numpy as jnp
from jax import lax
from jax.experimental import pallas as pl
from jax.experimental.pallas import tpu as pltpu
from jax.experimental.pallas import tpu_sc as plsc

_D = 1000
_CHUNK = 40


@functools.lru_cache(maxsize=None)
def _make_gather(n_idx: int, d: int):
    info = plsc.get_sparse_core_info()
    nc, ns = info.num_cores, info.num_subcores
    nw = nc * ns
    b_per_w = n_idx // nw
    assert n_idx % nw == 0 and b_per_w % _CHUNK == 0
    n_chunks = b_per_w // _CHUNK
    mesh = plsc.VectorSubcoreMesh(core_axis_name="c", subcore_axis_name="s")

    @functools.partial(
        pl.kernel,
        mesh=mesh,
        compiler_params=pltpu.CompilerParams(use_tc_tiling_on_sc=False),
        out_type=jax.ShapeDtypeStruct((n_idx, d), jnp.float32),
        scratch_types=[
            pltpu.VMEM((b_per_w,), jnp.int32),
            pltpu.VMEM((_CHUNK, d), jnp.float32),
            pltpu.SemaphoreType.DMA,
        ],
    )
    def k(idx_hbm, table_hbm, out_hbm, idx_v, rows_v, gsem):
        wid = lax.axis_index("s") * nc + lax.axis_index("c")
        base = wid * b_per_w
        pltpu.sync_copy(idx_hbm.at[pl.ds(base, b_per_w)], idx_v)

        def step(i, carry):
            off = i * _CHUNK
            pltpu.async_copy(
                table_hbm.at[idx_v.at[pl.ds(off, _CHUNK)]], rows_v, gsem
            ).wait()
            pltpu.sync_copy(rows_v, out_hbm.at[pl.ds(base + off, _CHUNK)])
            return carry

        lax.fori_loop(0, n_chunks, step, 0)

    return k


def kernel(idx, embedding_table):
    b, s = idx.shape
    v, d = embedding_table.shape
    flat = idx.reshape(-1).astype(jnp.int32)
    out = _make_gather(b * s, d)(flat, embedding_table)
    return out.reshape(b, s, d)


# speedup vs baseline: 1.0295x; 1.0295x over previous
import functools

import jax
import jax.numpy as jnp
from jax import lax
from jax.experimental import pallas as pl
from jax.experimental.pallas import tpu as pltpu
from jax.experimental.pallas import tpu_sc as plsc

_D = 1000
_CHUNK = 40


@functools.lru_cache(maxsize=None)
def _make_gather(n_idx: int, d: int):
    info = plsc.get_sparse_core_info()
    nc, ns = info.num_cores, info.num_subcores
    nw = nc * ns
    b_per_w = n_idx // nw
    assert n_idx % nw == 0 and b_per_w % _CHUNK == 0
    n_chunks = b_per_w // _CHUNK
    mesh = plsc.VectorSubcoreMesh(core_axis_name="c", subcore_axis_name="s")

    @functools.partial(
        pl.kernel,
        mesh=mesh,
        compiler_params=pltpu.CompilerParams(use_tc_tiling_on_sc=False),
        out_type=jax.ShapeDtypeStruct((n_idx, d), jnp.float32),
        scratch_types=[
            pltpu.VMEM((b_per_w,), jnp.int32),
            pltpu.VMEM((_CHUNK, d), jnp.float32),
            pltpu.VMEM((_CHUNK, d), jnp.float32),
            pltpu.SemaphoreType.DMA,
            pltpu.SemaphoreType.DMA,
            pltpu.SemaphoreType.DMA,
            pltpu.SemaphoreType.DMA,
        ],
    )
    def k(idx_hbm, table_hbm, out_hbm, idx_v, buf0, buf1, g0, g1, s0, s1):
        wid = lax.axis_index("s") * nc + lax.axis_index("c")
        base = wid * b_per_w
        pltpu.sync_copy(idx_hbm.at[pl.ds(base, b_per_w)], idx_v)

        def gather(i, buf, sem):
            pltpu.async_copy(
                table_hbm.at[idx_v.at[pl.ds(i * _CHUNK, _CHUNK)]], buf, sem
            )

        def wait_gather(buf, sem):
            pltpu.make_async_copy(
                table_hbm.at[idx_v.at[pl.ds(0, _CHUNK)]], buf, sem
            ).wait()

        def scatter(i, buf, sem):
            pltpu.async_copy(buf, out_hbm.at[pl.ds(base + i * _CHUNK, _CHUNK)], sem)

        def wait_scatter(buf, sem):
            pltpu.make_async_copy(buf, out_hbm.at[pl.ds(base, _CHUNK)], sem).wait()

        npairs = n_chunks // 2
        gather(0, buf0, g0)

        def pair(t, carry):
            i0 = 2 * t
            wait_gather(buf0, g0)
            scatter(i0, buf0, s0)

            @pl.when(t > 0)
            def _():
                wait_scatter(buf1, s1)

            gather(i0 + 1, buf1, g1)
            wait_gather(buf1, g1)
            scatter(i0 + 1, buf1, s1)

            @pl.when(t < npairs - 1)
            def _():
                wait_scatter(buf0, s0)
                gather(i0 + 2, buf0, g0)

            return carry

        lax.fori_loop(0, npairs, pair, 0)
        wait_scatter(buf0, s0)
        wait_scatter(buf1, s1)

    return k


def kernel(idx, embedding_table):
    b, s = idx.shape
    v, d = embedding_table.shape
    flat = idx.reshape(-1).astype(jnp.int32)
    out = _make_gather(b * s, d)(flat, embedding_table)
    return out.reshape(b, s, d)


# speedup vs baseline: 1.0313x; 1.0018x over previous
import functools

import jax
import jax.numpy as jnp
from jax import lax
from jax.experimental import pallas as pl
from jax.experimental.pallas import tpu as pltpu
from jax.experimental.pallas import tpu_sc as plsc


@functools.lru_cache(maxsize=None)
def _make_gather(n_b: int, n_t: int, d: int):
    info = plsc.get_sparse_core_info()
    nc, ns = info.num_cores, info.num_subcores
    nw = nc * ns
    b_per_w = n_b // nw
    assert n_b % nw == 0 and b_per_w % 2 == 0
    mesh = plsc.VectorSubcoreMesh(core_axis_name="c", subcore_axis_name="s")

    @functools.partial(
        pl.kernel,
        mesh=mesh,
        compiler_params=pltpu.CompilerParams(use_tc_tiling_on_sc=False),
        out_type=jax.ShapeDtypeStruct((n_b, n_t, d), jnp.float32),
        scratch_types=[
            pltpu.VMEM((b_per_w, n_t), jnp.int32),
            pltpu.VMEM((n_t, d), jnp.float32),
            pltpu.VMEM((n_t, d), jnp.float32),
            pltpu.SemaphoreType.DMA,
            pltpu.SemaphoreType.DMA,
            pltpu.SemaphoreType.DMA,
            pltpu.SemaphoreType.DMA,
        ],
    )
    def k(idx_hbm, table_hbm, out_hbm, idx_v, buf0, buf1, g0, g1, s0, s1):
        wid = lax.axis_index("s") * nc + lax.axis_index("c")
        b0 = wid * b_per_w
        pltpu.sync_copy(idx_hbm.at[pl.ds(b0, b_per_w)], idx_v)

        def gather(j, buf, sem):
            pltpu.async_copy(table_hbm.at[idx_v.at[j]], buf, sem)

        def wait_gather(buf, sem):
            pltpu.make_async_copy(table_hbm.at[idx_v.at[0]], buf, sem).wait()

        def scatter(j, buf, sem):
            pltpu.async_copy(buf, out_hbm.at[b0 + j], sem)

        def wait_scatter(buf, sem):
            pltpu.make_async_copy(buf, out_hbm.at[b0], sem).wait()

        npairs = b_per_w // 2
        gather(0, buf0, g0)

        def pair(t, carry):
            j0 = 2 * t
            wait_gather(buf0, g0)
            scatter(j0, buf0, s0)

            @pl.when(t > 0)
            def _():
                wait_scatter(buf1, s1)

            gather(j0 + 1, buf1, g1)
            wait_gather(buf1, g1)
            scatter(j0 + 1, buf1, s1)

            @pl.when(t < npairs - 1)
            def _():
                wait_scatter(buf0, s0)
                gather(j0 + 2, buf0, g0)

            return carry

        lax.fori_loop(0, npairs, pair, 0)
        wait_scatter(buf0, s0)
        wait_scatter(buf1, s1)

    return k


def kernel(idx, embedding_table):
    b, s = idx.shape
    v, d = embedding_table.shape
    return _make_gather(b, s, d)(idx.astype(jnp.int32), embedding_table)


# speedup vs baseline: 1.2078x; 1.1712x over previous
import functools

import jax
import jax.numpy as jnp
from jax import lax
from jax.experimental import pallas as pl
from jax.experimental.pallas import tpu as pltpu
from jax.experimental.pallas import tpu_sc as plsc


@functools.lru_cache(maxsize=None)
def _make_gather(n_b: int, n_t: int, d: int):
    info = plsc.get_sparse_core_info()
    nc, ns, nl = info.num_cores, info.num_subcores, info.num_lanes
    nw = nc * ns
    b_per_w = n_b // nw
    assert n_b % nw == 0 and b_per_w == 2 * nl and d % 8 == 0 and n_t % 2 == 0
    d8 = d // 8
    mesh = plsc.VectorSubcoreMesh(core_axis_name="c", subcore_axis_name="s")

    @functools.partial(
        pl.kernel,
        mesh=mesh,
        compiler_params=pltpu.CompilerParams(
            use_tc_tiling_on_sc=False, needs_layout_passes=False
        ),
        out_type=jax.ShapeDtypeStruct((n_t, d8, (n_b // 128) * 8, 128), jnp.float32),
        scratch_types=[
            pltpu.VMEM((n_t, b_per_w), jnp.int32),
            pltpu.VMEM((nl, d), jnp.float32),
            pltpu.VMEM((nl, d), jnp.float32),
            pltpu.VMEM((d8, 8, b_per_w), jnp.float32),
            pltpu.VMEM((d8, 8, b_per_w), jnp.float32),
            pltpu.SemaphoreType.DMA,
            pltpu.SemaphoreType.DMA,
            pltpu.SemaphoreType.DMA,
            pltpu.SemaphoreType.DMA,
        ],
    )
    def k(idx_t_hbm, table_hbm, out_hbm, idx_v, a0, a1, bb0, bb1, g0, g1, s0, s1):
        wid = lax.axis_index("s") * nc + lax.axis_index("c")
        bw0 = wid * b_per_w
        bh8 = (bw0 // 128) * 8
        bl0 = bw0 % 128
        iota = lax.broadcasted_iota(jnp.int32, (nl,), 0)

        pltpu.sync_copy(idx_t_hbm.at[:, pl.ds(bw0, b_per_w)], idx_v)

        def gather(t, h, abuf, sem):
            pltpu.async_copy(
                table_hbm.at[idx_v.at[t, pl.ds(nl * h, nl)]], abuf, sem
            )

        def wait_gather(abuf, sem):
            pltpu.make_async_copy(
                table_hbm.at[idx_v.at[0, pl.ds(0, nl)]], abuf, sem
            ).wait()

        def transpose(abuf, bbuf, h):
            def body(e_hi, c):
                for e_lo in range(8):
                    col = jnp.broadcast_to(e_hi * 8 + e_lo, (nl,))
                    v = plsc.load_gather(abuf, [iota, col])
                    bbuf[e_hi, e_lo, pl.ds(nl * h, nl)] = v
                return c

            lax.fori_loop(0, d8, body, 0)

        def write(t, bbuf, sem):
            pltpu.async_copy(
                bbuf, out_hbm.at[t, :, pl.ds(bh8, 8), pl.ds(bl0, b_per_w)], sem
            )

        def wait_write(bbuf, sem):
            pltpu.make_async_copy(
                bbuf, out_hbm.at[0, :, pl.ds(bh8, 8), pl.ds(bl0, b_per_w)], sem
            ).wait()

        npairs = n_t // 2
        gather(0, 0, a0, g0)

        def pair(tt, c):
            for sel, bbuf, sem in ((0, bb0, s0), (1, bb1, s1)):
                t = 2 * tt + sel
                wait_gather(a0, g0)
                gather(t, 1, a1, g1)

                @pl.when(tt > 0)
                def _():
                    wait_write(bbuf, sem)

                transpose(a0, bbuf, 0)
                wait_gather(a1, g1)
                if sel == 0:
                    gather(t + 1, 0, a0, g0)
                else:

                    @pl.when(tt < npairs - 1)
                    def _():
                        gather(t + 1, 0, a0, g0)

                transpose(a1, bbuf, 1)
                write(t, bbuf, sem)
            return c

        lax.fori_loop(0, npairs, pair, 0)
        wait_write(bb0, s0)
        wait_write(bb1, s1)

    return k


def kernel(idx, embedding_table):
    b, s = idx.shape
    v, d = embedding_table.shape
    idx_t = idx.T.astype(jnp.int32)
    phys = _make_gather(b, s, d)(idx_t, embedding_table)
    phys5 = phys.reshape(s, d // 8, b // 128, 8, 128)
    return phys5.transpose(2, 4, 0, 1, 3).reshape(b, s, d)


# speedup vs baseline: 3.1963x; 2.6464x over previous
import functools

import jax
import jax.numpy as jnp
from jax import lax
from jax.experimental import pallas as pl
from jax.experimental.pallas import tpu as pltpu
from jax.experimental.pallas import tpu_sc as plsc


@functools.lru_cache(maxsize=None)
def _make_gather(n_b: int, n_t: int, d: int):
    info = plsc.get_sparse_core_info()
    nc, ns, nl = info.num_cores, info.num_subcores, info.num_lanes
    nw = nc * ns
    b_per_w = n_b // nw
    assert n_b % nw == 0 and b_per_w == 2 * nl and d % 8 == 0 and n_t % 2 == 0
    d8 = d // 8
    mesh = plsc.VectorSubcoreMesh(core_axis_name="c", subcore_axis_name="s")

    @functools.partial(
        pl.kernel,
        mesh=mesh,
        compiler_params=pltpu.CompilerParams(
            use_tc_tiling_on_sc=False, needs_layout_passes=False
        ),
        out_type=jax.ShapeDtypeStruct((n_t, d8, (n_b // 128) * 8, 128), jnp.float32),
        scratch_types=[
            pltpu.VMEM((n_t, b_per_w), jnp.int32),
            pltpu.VMEM((nl, d), jnp.float32),
            pltpu.VMEM((nl, d), jnp.float32),
            pltpu.VMEM((d8, 8, b_per_w), jnp.float32),
            pltpu.VMEM((d8, 8, b_per_w), jnp.float32),
            pltpu.SemaphoreType.DMA,
            pltpu.SemaphoreType.DMA,
            pltpu.SemaphoreType.DMA,
            pltpu.SemaphoreType.DMA,
        ],
    )
    def k(idx_t_hbm, table_hbm, out_hbm, idx_v, a0, a1, bb0, bb1, g0, g1, s0, s1):
        wid = lax.axis_index("s") * nc + lax.axis_index("c")
        bw0 = wid * b_per_w
        bh8 = (bw0 // 128) * 8
        bl0 = bw0 % 128
        iota = lax.broadcasted_iota(jnp.int32, (nl,), 0)

        pltpu.sync_copy(idx_t_hbm.at[:, pl.ds(bw0, b_per_w)], idx_v)

        def gather(t, h, abuf, sem):
            pltpu.async_copy(
                table_hbm.at[idx_v.at[t, pl.ds(nl * h, nl)]], abuf, sem
            )

        def wait_gather(abuf, sem):
            pltpu.make_async_copy(
                table_hbm.at[idx_v.at[0, pl.ds(0, nl)]], abuf, sem
            ).wait()

        def transpose(abuf, bbuf, h):
            @plsc.parallel_loop(0, d8, 1, unroll=4)
            def body(e_hi):
                base = jnp.broadcast_to(e_hi * 8, (nl,))
                for e_lo in range(8):
                    v = plsc.load_gather(abuf, [iota, base + e_lo])
                    bbuf[e_hi, e_lo, pl.ds(nl * h, nl)] = v

        def write(t, bbuf, sem):
            pltpu.async_copy(
                bbuf, out_hbm.at[t, :, pl.ds(bh8, 8), pl.ds(bl0, b_per_w)], sem
            )

        def wait_write(bbuf, sem):
            pltpu.make_async_copy(
                bbuf, out_hbm.at[0, :, pl.ds(bh8, 8), pl.ds(bl0, b_per_w)], sem
            ).wait()

        npairs = n_t // 2
        gather(0, 0, a0, g0)

        def pair(tt, c):
            for sel, bbuf, sem in ((0, bb0, s0), (1, bb1, s1)):
                t = 2 * tt + sel
                wait_gather(a0, g0)
                gather(t, 1, a1, g1)

                @pl.when(tt > 0)
                def _():
                    wait_write(bbuf, sem)

                transpose(a0, bbuf, 0)
                wait_gather(a1, g1)
                if sel == 0:
                    gather(t + 1, 0, a0, g0)
                else:

                    @pl.when(tt < npairs - 1)
                    def _():
                        gather(t + 1, 0, a0, g0)

                transpose(a1, bbuf, 1)
                write(t, bbuf, sem)
            return c

        lax.fori_loop(0, npairs, pair, 0)
        wait_write(bb0, s0)
        wait_write(bb1, s1)

    return k


def kernel(idx, embedding_table):
    b, s = idx.shape
    v, d = embedding_table.shape
    idx_t = idx.T.astype(jnp.int32)
    phys = _make_gather(b, s, d)(idx_t, embedding_table)
    phys5 = phys.reshape(s, d // 8, b // 128, 8, 128)
    return phys5.transpose(2, 4, 0, 1, 3).reshape(b, s, d)
